# Initial kernel scaffold; baseline (speedup 1.0000x reference)
import jax
import jax.numpy as jnp
from jax.experimental import pallas as pl


def kernel(x, t):
    raise NotImplementedError("write your pallas kernel here")


# speedup vs baseline: 1.1776x; 1.1776x over previous
import functools

import jax
import jax.numpy as jnp
from jax import lax
from jax.experimental import pallas as pl
from jax.experimental.pallas import tpu as pltpu
from jax.experimental.pallas import tpu_sc as plsc


def _gather_call(V: int, D: int, N: int):
  info = plsc.get_sparse_core_info()
  NC, NS = info.num_cores, info.num_subcores
  NW = NC * NS
  assert N % NW == 0
  n_per_w = N // NW
  CHUNK = 128
  assert n_per_w % CHUNK == 0
  n_chunks = n_per_w // CHUNK
  K = 4
  G = K * CHUNK
  assert n_chunks % K == 0
  n_grp = n_chunks // K

  mesh = plsc.VectorSubcoreMesh(core_axis_name="c", subcore_axis_name="s")

  @functools.partial(
      pl.kernel,
      out_type=jax.ShapeDtypeStruct((N, D), jnp.float32),
      mesh=mesh,
      compiler_params=pltpu.CompilerParams(use_tc_tiling_on_sc=False),
      scratch_types=[
          pltpu.VMEM((n_chunks, CHUNK), jnp.int32),
          pltpu.VMEM((G, D), jnp.float32),
          pltpu.SemaphoreType.DMA,
      ],
  )
  def gather_kernel(table_hbm, idx_hbm, out_hbm, idx_v, rows_v, gsem):
    wid = lax.axis_index("s") * NC + lax.axis_index("c")
    base = wid * n_per_w
    pltpu.sync_copy(idx_hbm.at[wid], idx_v)

    @pl.loop(0, n_grp)
    def _(g):
      cps = []
      for j in range(K):
        cps.append(
            pltpu.async_copy(
                table_hbm.at[idx_v.at[g * K + j]],
                rows_v.at[pl.ds(j * CHUNK, CHUNK)],
                gsem,
            )
        )
      for cp in cps:
        cp.wait()
      pltpu.sync_copy(rows_v, out_hbm.at[pl.ds(base + g * G, G)])

  return gather_kernel, NW, n_chunks, CHUNK


def kernel(x, t):
  V, D = x.shape
  B, F = t.shape
  N = B * F
  call, NW, n_chunks, CHUNK = _gather_call(V, D, N)
  idx = t.reshape(NW, n_chunks, CHUNK).astype(jnp.int32)
  out = call(x, idx)
  return out.reshape(B, F * D)


# speedup vs baseline: 1.1961x; 1.0157x over previous
import functools

import jax
import jax.numpy as jnp
from jax import lax
from jax.experimental import pallas as pl
from jax.experimental.pallas import tpu as pltpu
from jax.experimental.pallas import tpu_sc as plsc


def _gather_call(V: int, D: int, N: int):
  info = plsc.get_sparse_core_info()
  NC, NS = info.num_cores, info.num_subcores
  NW = NC * NS
  assert N % NW == 0
  n_per_w = N // NW
  CHUNK = 128
  assert n_per_w % CHUNK == 0
  n_chunks = n_per_w // CHUNK
  K = 4
  G = K * CHUNK
  assert n_chunks % K == 0
  n_grp = n_chunks // K

  mesh = plsc.VectorSubcoreMesh(core_axis_name="c", subcore_axis_name="s")

  @functools.partial(
      pl.kernel,
      out_type=jax.ShapeDtypeStruct((N, D), jnp.float32),
      mesh=mesh,
      compiler_params=pltpu.CompilerParams(use_tc_tiling_on_sc=False),
      scratch_types=[
          pltpu.VMEM((n_chunks, CHUNK), jnp.int32),
          pltpu.VMEM((2, G, D), jnp.float32),
          pltpu.SemaphoreType.DMA,
          pltpu.SemaphoreType.DMA,
      ],
  )
  def gather_kernel(table_hbm, idx_hbm, out_hbm, idx_v, rows_v, gsem, osem):
    wid = lax.axis_index("s") * NC + lax.axis_index("c")
    base = wid * n_per_w
    pltpu.sync_copy(idx_hbm.at[wid], idx_v)

    def fire(g, buf):
      for j in range(K):
        pltpu.async_copy(
            table_hbm.at[idx_v.at[g * K + j]],
            rows_v.at[buf, pl.ds(j * CHUNK, CHUNK)],
            gsem,
        )

    fire(0, 0)

    @pl.loop(0, n_grp, step=2)
    def _(g0):
      for s in range(2):
        g = g0 + s
        pltpu.make_async_copy(
            table_hbm.at[pl.ds(0, G)], rows_v.at[s], gsem
        ).wait()
        out_cp = pltpu.async_copy(
            rows_v.at[s], out_hbm.at[pl.ds(base + g * G, G)], osem
        )
        @pl.when(g >= 1)
        def _():
          pltpu.make_async_copy(
              table_hbm.at[pl.ds(0, G)], rows_v.at[1 - s], osem
          ).wait()

        @pl.when(g + 1 < n_grp)
        def _():
          fire(g + 1, 1 - s)

    pltpu.make_async_copy(
        table_hbm.at[pl.ds(0, G)], rows_v.at[0], osem
    ).wait()

  return gather_kernel, NW, n_chunks, CHUNK


def kernel(x, t):
  V, D = x.shape
  B, F = t.shape
  N = B * F
  call, NW, n_chunks, CHUNK = _gather_call(V, D, N)
  idx = t.reshape(NW, n_chunks, CHUNK).astype(jnp.int32)
  out = call(x, idx)
  return out.reshape(B, F * D)


# speedup vs baseline: 1.2022x; 1.0051x over previous
import functools

import jax
import jax.numpy as jnp
from jax import lax
from jax.experimental import pallas as pl
from jax.experimental.pallas import tpu as pltpu
from jax.experimental.pallas import tpu_sc as plsc


def _gather_call(V: int, D: int, N: int):
  info = plsc.get_sparse_core_info()
  NC, NS = info.num_cores, info.num_subcores
  NW = NC * NS
  assert N % NW == 0
  n_per_w = N // NW
  CHUNK = 128
  assert n_per_w % CHUNK == 0
  n_chunks = n_per_w // CHUNK
  RING = 8
  GDEPTH = 4
  assert n_chunks % RING == 0

  mesh = plsc.VectorSubcoreMesh(core_axis_name="c", subcore_axis_name="s")

  @functools.partial(
      pl.kernel,
      out_type=jax.ShapeDtypeStruct((N, D), jnp.float32),
      mesh=mesh,
      compiler_params=pltpu.CompilerParams(use_tc_tiling_on_sc=False),
      scratch_types=[
          pltpu.VMEM((n_chunks, CHUNK), jnp.int32),
          pltpu.VMEM((RING * CHUNK, D), jnp.float32),
      ] + [pltpu.SemaphoreType.DMA] * (2 * RING),
  )
  def gather_kernel(table_hbm, idx_hbm, out_hbm, idx_v, rows_v, *sems):
    gsem, osem = sems[:RING], sems[RING:]
    wid = lax.axis_index("s") * NC + lax.axis_index("c")
    base = wid * n_per_w
    pltpu.sync_copy(idx_hbm.at[wid], idx_v)

    def slot(s):
      return rows_v.at[pl.ds(s * CHUNK, CHUNK)]

    def fire(c, s):
      pltpu.async_copy(table_hbm.at[idx_v.at[c]], slot(s), gsem[s])

    def drain(s, sem):
      pltpu.make_async_copy(table_hbm.at[pl.ds(0, CHUNK)], slot(s), sem).wait()

    for s in range(GDEPTH):
      fire(s, s)

    @pl.loop(0, n_chunks, step=RING)
    def _(c0):
      for k in range(RING):
        c = c0 + k
        drain(k, gsem[k])
        pltpu.async_copy(
            slot(k), out_hbm.at[pl.ds(base + c * CHUNK, CHUNK)], osem[k]
        )
        s2 = (k + GDEPTH) % RING

        @pl.when(c >= GDEPTH)
        def _():
          drain(s2, osem[s2])

        @pl.when(c + GDEPTH < n_chunks)
        def _():
          fire(c + GDEPTH, s2)

    for s in range(GDEPTH, RING):
      drain(s, osem[s])

  return gather_kernel, NW, n_chunks, CHUNK


def kernel(x, t):
  V, D = x.shape
  B, F = t.shape
  N = B * F
  call, NW, n_chunks, CHUNK = _gather_call(V, D, N)
  idx = t.reshape(NW, n_chunks, CHUNK).astype(jnp.int32)
  out = call(x, idx)
  return out.reshape(B, F * D)
